# baseline (speedup 1.0000x reference)
import functools

import jax
import jax.numpy as jnp
from jax import lax
from jax.experimental import pallas as pl
from jax.experimental.pallas import tpu as pltpu
from jax.experimental.pallas import tpu_sc as plsc

N = 10000
NPAD = 10240
E = 160000
H = 256
HP = H // 2
G = 128

BN = 2048
NB = NPAD // BN

RPK = 4
NTILES = 32
CHUNK = 1280
NCH = E // CHUNK
GRP = CHUNK // 16

_DN_NT = (((1,), (1,)), ((), ()))
_DN_NN = (((1,), (0,)), ((), ()))


def _pack_rows(even_f32, odd_f32):
    pe = lax.bitcast_convert_type(even_f32.astype(jnp.bfloat16), jnp.uint16)
    po = lax.bitcast_convert_type(odd_f32.astype(jnp.bfloat16), jnp.uint16)
    word = pe.astype(jnp.uint32) | (po.astype(jnp.uint32) << 16)
    return lax.bitcast_convert_type(word, jnp.int32)


def _dense1pk_body(we_ref, wo_ref, x_ref, pk_ref):
    x = x_ref[...].astype(jnp.bfloat16)
    ye = lax.dot_general(we_ref[...], x, _DN_NT,
                         preferred_element_type=jnp.float32)
    yo = lax.dot_general(wo_ref[...], x, _DN_NT,
                         preferred_element_type=jnp.float32)
    pk_ref[...] = _pack_rows(ye, yo)


def _dense1r_body(wroot_ref, b_ref, x_ref, r_ref):
    x = x_ref[...].astype(jnp.bfloat16)
    r_ref[...] = lax.dot_general(wroot_ref[...], x, _DN_NT,
                                 preferred_element_type=jnp.float32) + b_ref[...]


def _dense2pk_body(we_ref, wo_ref, agg_ref, r_ref, pk_ref):
    h = jnp.maximum(agg_ref[...] + r_ref[...], 0.0).astype(jnp.bfloat16)
    ye = lax.dot_general(we_ref[...], h, _DN_NN,
                         preferred_element_type=jnp.float32)
    yo = lax.dot_general(wo_ref[...], h, _DN_NN,
                         preferred_element_type=jnp.float32)
    pk_ref[...] = _pack_rows(ye, yo)


def _dense2r_body(wroot_ref, b_ref, agg_ref, r_ref, r2_ref):
    h = jnp.maximum(agg_ref[...] + r_ref[...], 0.0).astype(jnp.bfloat16)
    r2_ref[...] = lax.dot_general(wroot_ref[...], h, _DN_NN,
                                  preferred_element_type=jnp.float32) + b_ref[...]


def _pool_mlp_body(wp_ref, bp_ref, wf_ref, bf_ref, agg_ref, r_ref, batch_ref,
                   o_ref, sums_ref, cnt_ref):
    k = pl.program_id(0)

    @pl.when(k == 0)
    def _():
        sums_ref[...] = jnp.zeros_like(sums_ref)
        cnt_ref[...] = jnp.zeros_like(cnt_ref)

    h = jnp.maximum(agg_ref[...] + r_ref[...], 0.0)
    bm = batch_ref[0]
    h = jnp.where(bm >= 0, h, 0.0)
    b = bm.reshape(BN, 1)
    gids = lax.broadcasted_iota(jnp.int32, (BN, G), 1)
    oh = jnp.where(b == gids, 1.0, 0.0)
    sums_ref[...] += lax.dot_general(h, oh, _DN_NN,
                                     preferred_element_type=jnp.float32)
    cnt_ref[...] += jnp.sum(oh, axis=0, keepdims=True)

    @pl.when(k == NB - 1)
    def _():
        pooled = sums_ref[...] / jnp.maximum(cnt_ref[...], 1.0)
        z = lax.dot_general(wp_ref[...], pooled, _DN_NN,
                            preferred_element_type=jnp.float32) + bp_ref[...]
        z = jnp.maximum(z, 0.0)
        o = lax.dot_general(wf_ref[...], z, (((0,), (0,)), ((), ())),
                            preferred_element_type=jnp.float32) + bf_ref[...]
        o_ref[...] = jnp.broadcast_to(1.0 / (1.0 + jnp.exp(-o)), (8, G))


def _full(shape):
    return pl.BlockSpec(shape, lambda i: tuple(0 for _ in shape))


_dense1pk = pl.pallas_call(
    _dense1pk_body,
    grid=(NB,),
    in_specs=[
        _full((HP, H)), _full((HP, H)),
        pl.BlockSpec((BN, H), lambda i: (i, 0)),
    ],
    out_specs=pl.BlockSpec((HP, BN), lambda i: (0, i)),
    out_shape=jax.ShapeDtypeStruct((HP, NPAD), jnp.int32),
)

_dense1r = pl.pallas_call(
    _dense1r_body,
    grid=(NB,),
    in_specs=[
        _full((H, H)), _full((H, 1)),
        pl.BlockSpec((BN, H), lambda i: (i, 0)),
    ],
    out_specs=pl.BlockSpec((H, BN), lambda i: (0, i)),
    out_shape=jax.ShapeDtypeStruct((H, NPAD), jnp.float32),
)

_dense2pk = pl.pallas_call(
    _dense2pk_body,
    grid=(NB,),
    in_specs=[
        _full((HP, H)), _full((HP, H)),
        pl.BlockSpec((H, BN), lambda i: (0, i)),
        pl.BlockSpec((H, BN), lambda i: (0, i)),
    ],
    out_specs=pl.BlockSpec((HP, BN), lambda i: (0, i)),
    out_shape=jax.ShapeDtypeStruct((HP, NPAD), jnp.int32),
)

_dense2r = pl.pallas_call(
    _dense2r_body,
    grid=(NB,),
    in_specs=[
        _full((H, H)), _full((H, 1)),
        pl.BlockSpec((H, BN), lambda i: (0, i)),
        pl.BlockSpec((H, BN), lambda i: (0, i)),
    ],
    out_specs=pl.BlockSpec((H, BN), lambda i: (0, i)),
    out_shape=jax.ShapeDtypeStruct((H, NPAD), jnp.float32),
)

_pool_mlp = pl.pallas_call(
    _pool_mlp_body,
    grid=(NB,),
    in_specs=[
        _full((H, H)), _full((H, 1)), _full((H, 1)), _full((1, 1)),
        pl.BlockSpec((H, BN), lambda i: (0, i)),
        pl.BlockSpec((H, BN), lambda i: (0, i)),
        pl.BlockSpec((1, 1, BN), lambda i: (i, 0, 0)),
    ],
    out_specs=pl.BlockSpec((8, G), lambda i: (0, 0)),
    out_shape=jax.ShapeDtypeStruct((8, G), jnp.float32),
    scratch_shapes=[
        pltpu.VMEM((H, G), jnp.float32),
        pltpu.VMEM((1, G), jnp.float32),
    ],
)


def _make_segsum():
    mesh = plsc.VectorSubcoreMesh(core_axis_name="c", subcore_axis_name="s")

    @functools.partial(
        pl.kernel,
        out_type=jax.ShapeDtypeStruct((H, NPAD), jnp.float32),
        mesh=mesh,
        compiler_params=pltpu.CompilerParams(
            use_tc_tiling_on_sc=False, needs_layout_passes=False),
        scratch_types=[
            pltpu.VMEM((RPK, NPAD), jnp.int32),
            pltpu.VMEM((2 * RPK, NPAD), jnp.float32),
            pltpu.VMEM((CHUNK,), jnp.int32),
            pltpu.VMEM((CHUNK,), jnp.int32),
            pltpu.VMEM((CHUNK,), jnp.int32),
            pltpu.VMEM((CHUNK,), jnp.int32),
            pltpu.VMEM((CHUNK,), jnp.float32),
            pltpu.VMEM((CHUNK,), jnp.float32),
            pltpu.SemaphoreType.DMA,
            pltpu.SemaphoreType.DMA,
            pltpu.SemaphoreType.DMA,
            pltpu.SemaphoreType.DMA,
            pltpu.SemaphoreType.DMA,
            pltpu.SemaphoreType.DMA,
            pltpu.SemaphoreType.DMA,
        ],
    )
    def seg(pk_hbm, src_hbm, dst_hbm, w_hbm, out_hbm,
            table, acc, s_a, s_b, d_a, d_b, w_a, w_b,
            sem_sa, sem_sb, sem_da, sem_db, sem_wa, sem_wb, sem_tb):
        cid = lax.axis_index("c")
        sid = lax.axis_index("s")
        wid = sid * 2 + cid

        def start(g, sbuf, dbuf, wbuf, sem_s, sem_d, sem_w):
            off = g * CHUNK
            pltpu.async_copy(src_hbm.at[pl.ds(off, CHUNK)], sbuf, sem_s)
            pltpu.async_copy(dst_hbm.at[pl.ds(off, CHUNK)], dbuf, sem_d)
            pltpu.async_copy(w_hbm.at[pl.ds(off, CHUNK)], wbuf, sem_w)

        def wait(sbuf, dbuf, wbuf, sem_s, sem_d, sem_w):
            pltpu.make_async_copy(src_hbm.at[pl.ds(0, CHUNK)], sbuf, sem_s).wait()
            pltpu.make_async_copy(dst_hbm.at[pl.ds(0, CHUNK)], dbuf, sem_d).wait()
            pltpu.make_async_copy(w_hbm.at[pl.ds(0, CHUNK)], wbuf, sem_w).wait()

        def process(sbuf, dbuf, wbuf):
            @plsc.parallel_loop(0, CHUNK, step=16, unroll=4)
            def gbody(base):
                sv = sbuf[pl.ds(base, 16)]
                dv = dbuf[pl.ds(base, 16)]
                wv = wbuf[pl.ds(base, 16)]
                for c in range(RPK):
                    cv = jnp.full((16,), c, jnp.int32)
                    gw = plsc.load_gather(table, [cv, sv])
                    bf = plsc.bitcast(gw, jnp.bfloat16)
                    lo, hi = plsc.unpack(bf, format=plsc.PackFormat.INTERLEAVED,
                                         preferred_element_type=jnp.float32)
                    plsc.addupdate_scatter(
                        acc, [jnp.full((16,), 2 * c, jnp.int32), dv], lo * wv)
                    plsc.addupdate_scatter(
                        acc, [jnp.full((16,), 2 * c + 1, jnp.int32), dv], hi * wv)

        r0 = wid * RPK
        tbl_cp = pltpu.async_copy(pk_hbm.at[pl.ds(r0, RPK), :], table, sem_tb)

        zeros16 = jnp.zeros((16,), jnp.float32)

        @plsc.parallel_loop(0, NPAD, step=32, unroll=4)
        def zbody(i):
            for r in range(2 * RPK):
                for u in range(2):
                    acc[r, pl.ds(i + u * 16, 16)] = zeros16

        tbl_cp.wait()

        start(0, s_a, d_a, w_a, sem_sa, sem_da, sem_wa)
        start(1, s_b, d_b, w_b, sem_sb, sem_db, sem_wb)

        def chunk_body(g2, carry):
            ga = g2 * 2
            wait(s_a, d_a, w_a, sem_sa, sem_da, sem_wa)
            process(s_a, d_a, w_a)

            @pl.when(ga + 2 < NCH)
            def _():
                start(ga + 2, s_a, d_a, w_a, sem_sa, sem_da, sem_wa)

            wait(s_b, d_b, w_b, sem_sb, sem_db, sem_wb)
            process(s_b, d_b, w_b)

            @pl.when(ga + 3 < NCH)
            def _():
                start(ga + 3, s_b, d_b, w_b, sem_sb, sem_db, sem_wb)
            return carry

        lax.fori_loop(0, NCH // 2, chunk_body, 0)
        if NCH % 2:
            wait(s_a, d_a, w_a, sem_sa, sem_da, sem_wa)
            process(s_a, d_a, w_a)

        pltpu.sync_copy(acc, out_hbm.at[pl.ds(wid * 2 * RPK, 2 * RPK), :])

    return seg


@functools.cache
def _get_segsum():
    return _make_segsum()


def kernel(x_static_graph, edge_index, edge_weight, batch, target_index,
           W1_rel, b1, W1_root, W2_rel, b2, W2_root, Wp, bp, Wf, bf):
    batch_pad = jnp.pad(batch, (0, NPAD - N), constant_values=-1)
    batch3d = batch_pad.reshape(NB, 1, BN)
    src = edge_index[0]
    dst = edge_index[1]

    segsum = _get_segsum()
    bft = jnp.bfloat16
    y1pk = _dense1pk(W1_rel[0::2].astype(bft), W1_rel[1::2].astype(bft),
                     x_static_graph)
    agg1 = segsum(y1pk, src, dst, edge_weight)
    r1 = _dense1r(W1_root.astype(bft), b1.reshape(H, 1), x_static_graph)
    y2pk = _dense2pk(W2_rel[0::2].astype(bft), W2_rel[1::2].astype(bft),
                     agg1, r1)
    agg2 = segsum(y2pk, src, dst, edge_weight)
    r2 = _dense2r(W2_root.astype(bft), b2.reshape(H, 1), agg1, r1)
    o = _pool_mlp(Wp, bp.reshape(H, 1), Wf.reshape(H, 1), bf.reshape(1, 1),
                  agg2, r2, batch3d)
    return o[0:1, :].reshape(G, 1)


# speedup vs baseline: 1.0157x; 1.0157x over previous
import functools

import jax
import jax.numpy as jnp
from jax import lax
from jax.experimental import pallas as pl
from jax.experimental.pallas import tpu as pltpu
from jax.experimental.pallas import tpu_sc as plsc

N = 10000
NPAD = 10240
E = 160000
H = 256
HP = H // 2
G = 128

BN = 2048
NB = NPAD // BN

RPK = 4
NTILES = 32
CHUNK = 1280
NCH = E // CHUNK
GRP = CHUNK // 16

_DN_NT = (((1,), (1,)), ((), ()))
_DN_NN = (((1,), (0,)), ((), ()))


def _pack_rows(even_f32, odd_f32):
    pe = lax.bitcast_convert_type(even_f32.astype(jnp.bfloat16), jnp.uint16)
    po = lax.bitcast_convert_type(odd_f32.astype(jnp.bfloat16), jnp.uint16)
    word = pe.astype(jnp.uint32) | (po.astype(jnp.uint32) << 16)
    return lax.bitcast_convert_type(word, jnp.int32)


def _dense1_body(we_ref, wo_ref, wroot_ref, b_ref, x_ref, pk_ref, r_ref):
    x = x_ref[...].astype(jnp.bfloat16)
    ye = lax.dot_general(we_ref[...], x, _DN_NT,
                         preferred_element_type=jnp.float32)
    yo = lax.dot_general(wo_ref[...], x, _DN_NT,
                         preferred_element_type=jnp.float32)
    pk_ref[...] = _pack_rows(ye, yo)
    r_ref[...] = lax.dot_general(wroot_ref[...], x, _DN_NT,
                                 preferred_element_type=jnp.float32) + b_ref[...]


def _dense2_body(we_ref, wo_ref, wroot_ref, b_ref, agg_ref, r_ref,
                 pk_ref, r2_ref):
    h = jnp.maximum(agg_ref[...] + r_ref[...], 0.0).astype(jnp.bfloat16)
    ye = lax.dot_general(we_ref[...], h, _DN_NN,
                         preferred_element_type=jnp.float32)
    yo = lax.dot_general(wo_ref[...], h, _DN_NN,
                         preferred_element_type=jnp.float32)
    pk_ref[...] = _pack_rows(ye, yo)
    r2_ref[...] = lax.dot_general(wroot_ref[...], h, _DN_NN,
                                  preferred_element_type=jnp.float32) + b_ref[...]


def _pool_mlp_body(wp_ref, bp_ref, wf_ref, bf_ref, agg_ref, r_ref, batch_ref,
                   o_ref, sums_ref, cnt_ref):
    k = pl.program_id(0)

    @pl.when(k == 0)
    def _():
        sums_ref[...] = jnp.zeros_like(sums_ref)
        cnt_ref[...] = jnp.zeros_like(cnt_ref)

    h = jnp.maximum(agg_ref[...] + r_ref[...], 0.0)
    bm = batch_ref[0]
    h = jnp.where(bm >= 0, h, 0.0)
    b = bm.reshape(BN, 1)
    gids = lax.broadcasted_iota(jnp.int32, (BN, G), 1)
    oh = jnp.where(b == gids, 1.0, 0.0)
    sums_ref[...] += lax.dot_general(h, oh, _DN_NN,
                                     preferred_element_type=jnp.float32)
    cnt_ref[...] += jnp.sum(oh, axis=0, keepdims=True)

    @pl.when(k == NB - 1)
    def _():
        pooled = sums_ref[...] / jnp.maximum(cnt_ref[...], 1.0)
        z = lax.dot_general(wp_ref[...], pooled, _DN_NN,
                            preferred_element_type=jnp.float32) + bp_ref[...]
        z = jnp.maximum(z, 0.0)
        o = lax.dot_general(wf_ref[...], z, (((0,), (0,)), ((), ())),
                            preferred_element_type=jnp.float32) + bf_ref[...]
        o_ref[...] = jnp.broadcast_to(1.0 / (1.0 + jnp.exp(-o)), (8, G))


def _full(shape):
    return pl.BlockSpec(shape, lambda i: tuple(0 for _ in shape))


_dense1 = pl.pallas_call(
    _dense1_body,
    grid=(NB,),
    in_specs=[
        _full((HP, H)), _full((HP, H)), _full((H, H)), _full((H, 1)),
        pl.BlockSpec((BN, H), lambda i: (i, 0)),
    ],
    out_specs=[
        pl.BlockSpec((HP, BN), lambda i: (0, i)),
        pl.BlockSpec((H, BN), lambda i: (0, i)),
    ],
    out_shape=[
        jax.ShapeDtypeStruct((HP, NPAD), jnp.int32),
        jax.ShapeDtypeStruct((H, NPAD), jnp.float32),
    ],
)

_dense2 = pl.pallas_call(
    _dense2_body,
    grid=(NB,),
    in_specs=[
        _full((HP, H)), _full((HP, H)), _full((H, H)), _full((H, 1)),
        pl.BlockSpec((H, BN), lambda i: (0, i)),
        pl.BlockSpec((H, BN), lambda i: (0, i)),
    ],
    out_specs=[
        pl.BlockSpec((HP, BN), lambda i: (0, i)),
        pl.BlockSpec((H, BN), lambda i: (0, i)),
    ],
    out_shape=[
        jax.ShapeDtypeStruct((HP, NPAD), jnp.int32),
        jax.ShapeDtypeStruct((H, NPAD), jnp.float32),
    ],
)

_pool_mlp = pl.pallas_call(
    _pool_mlp_body,
    grid=(NB,),
    in_specs=[
        _full((H, H)), _full((H, 1)), _full((H, 1)), _full((1, 1)),
        pl.BlockSpec((H, BN), lambda i: (0, i)),
        pl.BlockSpec((H, BN), lambda i: (0, i)),
        pl.BlockSpec((1, 1, BN), lambda i: (i, 0, 0)),
    ],
    out_specs=pl.BlockSpec((8, G), lambda i: (0, 0)),
    out_shape=jax.ShapeDtypeStruct((8, G), jnp.float32),
    scratch_shapes=[
        pltpu.VMEM((H, G), jnp.float32),
        pltpu.VMEM((1, G), jnp.float32),
    ],
)


def _make_segsum():
    mesh = plsc.VectorSubcoreMesh(core_axis_name="c", subcore_axis_name="s")

    @functools.partial(
        pl.kernel,
        out_type=jax.ShapeDtypeStruct((H, NPAD), jnp.float32),
        mesh=mesh,
        compiler_params=pltpu.CompilerParams(
            use_tc_tiling_on_sc=False, needs_layout_passes=False),
        scratch_types=[
            pltpu.VMEM((RPK, NPAD), jnp.int32),
            pltpu.VMEM((2 * RPK, NPAD), jnp.float32),
            pltpu.VMEM((CHUNK,), jnp.int32),
            pltpu.VMEM((CHUNK,), jnp.int32),
            pltpu.VMEM((CHUNK,), jnp.int32),
            pltpu.VMEM((CHUNK,), jnp.int32),
            pltpu.VMEM((CHUNK,), jnp.float32),
            pltpu.VMEM((CHUNK,), jnp.float32),
            pltpu.SemaphoreType.DMA,
            pltpu.SemaphoreType.DMA,
            pltpu.SemaphoreType.DMA,
            pltpu.SemaphoreType.DMA,
            pltpu.SemaphoreType.DMA,
            pltpu.SemaphoreType.DMA,
            pltpu.SemaphoreType.DMA,
        ],
    )
    def seg(pk_hbm, src_hbm, dst_hbm, w_hbm, out_hbm,
            table, acc, s_a, s_b, d_a, d_b, w_a, w_b,
            sem_sa, sem_sb, sem_da, sem_db, sem_wa, sem_wb, sem_tb):
        cid = lax.axis_index("c")
        sid = lax.axis_index("s")
        wid = sid * 2 + cid

        def start(g, sbuf, dbuf, wbuf, sem_s, sem_d, sem_w):
            off = g * CHUNK
            pltpu.async_copy(src_hbm.at[pl.ds(off, CHUNK)], sbuf, sem_s)
            pltpu.async_copy(dst_hbm.at[pl.ds(off, CHUNK)], dbuf, sem_d)
            pltpu.async_copy(w_hbm.at[pl.ds(off, CHUNK)], wbuf, sem_w)

        def wait(sbuf, dbuf, wbuf, sem_s, sem_d, sem_w):
            pltpu.make_async_copy(src_hbm.at[pl.ds(0, CHUNK)], sbuf, sem_s).wait()
            pltpu.make_async_copy(dst_hbm.at[pl.ds(0, CHUNK)], dbuf, sem_d).wait()
            pltpu.make_async_copy(w_hbm.at[pl.ds(0, CHUNK)], wbuf, sem_w).wait()

        def process(sbuf, dbuf, wbuf):
            @plsc.parallel_loop(0, CHUNK, step=16, unroll=4)
            def gbody(base):
                sv = sbuf[pl.ds(base, 16)]
                dv = dbuf[pl.ds(base, 16)]
                wv = wbuf[pl.ds(base, 16)]
                for c in range(RPK):
                    cv = jnp.full((16,), c, jnp.int32)
                    gw = plsc.load_gather(table, [cv, sv])
                    bf = plsc.bitcast(gw, jnp.bfloat16)
                    lo, hi = plsc.unpack(bf, format=plsc.PackFormat.INTERLEAVED,
                                         preferred_element_type=jnp.float32)
                    plsc.addupdate_scatter(
                        acc, [jnp.full((16,), 2 * c, jnp.int32), dv], lo * wv)
                    plsc.addupdate_scatter(
                        acc, [jnp.full((16,), 2 * c + 1, jnp.int32), dv], hi * wv)

        r0 = wid * RPK
        tbl_cp = pltpu.async_copy(pk_hbm.at[pl.ds(r0, RPK), :], table, sem_tb)

        zeros16 = jnp.zeros((16,), jnp.float32)

        @plsc.parallel_loop(0, NPAD, step=32, unroll=4)
        def zbody(i):
            for r in range(2 * RPK):
                for u in range(2):
                    acc[r, pl.ds(i + u * 16, 16)] = zeros16

        tbl_cp.wait()

        start(0, s_a, d_a, w_a, sem_sa, sem_da, sem_wa)
        start(1, s_b, d_b, w_b, sem_sb, sem_db, sem_wb)

        def chunk_body(g2, carry):
            ga = g2 * 2
            wait(s_a, d_a, w_a, sem_sa, sem_da, sem_wa)
            process(s_a, d_a, w_a)

            @pl.when(ga + 2 < NCH)
            def _():
                start(ga + 2, s_a, d_a, w_a, sem_sa, sem_da, sem_wa)

            wait(s_b, d_b, w_b, sem_sb, sem_db, sem_wb)
            process(s_b, d_b, w_b)

            @pl.when(ga + 3 < NCH)
            def _():
                start(ga + 3, s_b, d_b, w_b, sem_sb, sem_db, sem_wb)
            return carry

        lax.fori_loop(0, NCH // 2, chunk_body, 0)
        if NCH % 2:
            wait(s_a, d_a, w_a, sem_sa, sem_da, sem_wa)
            process(s_a, d_a, w_a)

        pltpu.sync_copy(acc, out_hbm.at[pl.ds(wid * 2 * RPK, 2 * RPK), :])

    return seg


@functools.cache
def _get_segsum():
    return _make_segsum()


def kernel(x_static_graph, edge_index, edge_weight, batch, target_index,
           W1_rel, b1, W1_root, W2_rel, b2, W2_root, Wp, bp, Wf, bf):
    batch_pad = jnp.pad(batch, (0, NPAD - N), constant_values=-1)
    batch3d = batch_pad.reshape(NB, 1, BN)
    src = edge_index[0]
    dst = edge_index[1]

    segsum = _get_segsum()
    bft = jnp.bfloat16
    y1pk, r1 = _dense1(W1_rel[0::2].astype(bft), W1_rel[1::2].astype(bft),
                       W1_root.astype(bft), b1.reshape(H, 1), x_static_graph)
    agg1 = segsum(y1pk, src, dst, edge_weight)
    y2pk, r2 = _dense2(W2_rel[0::2].astype(bft), W2_rel[1::2].astype(bft),
                       W2_root.astype(bft), b2.reshape(H, 1), agg1, r1)
    agg2 = segsum(y2pk, src, dst, edge_weight)
    o = _pool_mlp(Wp, bp.reshape(H, 1), Wf.reshape(H, 1), bf.reshape(1, 1),
                  agg2, r2, batch3d)
    return o[0:1, :].reshape(G, 1)


# speedup vs baseline: 1.0603x; 1.0440x over previous
import functools

import jax
import jax.numpy as jnp
from jax import lax
from jax.experimental import pallas as pl
from jax.experimental.pallas import tpu as pltpu
from jax.experimental.pallas import tpu_sc as plsc

N = 10000
NPAD = 10240
E = 160000
H = 256
HP = H // 2
G = 128

BN = 2048
NB = NPAD // BN

RPK = 4
NTILES = 32
CHUNK = 1600
NCH = E // CHUNK
GRP = CHUNK // 16

_DN_NT = (((1,), (1,)), ((), ()))
_DN_NN = (((1,), (0,)), ((), ()))


def _pack_rows(even_f32, odd_f32):
    pe = lax.bitcast_convert_type(even_f32.astype(jnp.bfloat16), jnp.uint16)
    po = lax.bitcast_convert_type(odd_f32.astype(jnp.bfloat16), jnp.uint16)
    word = pe.astype(jnp.uint32) | (po.astype(jnp.uint32) << 16)
    return lax.bitcast_convert_type(word, jnp.int32)


def _dense1_body(we_ref, wo_ref, wroot_ref, b_ref, x_ref, pk_ref, r_ref):
    x = x_ref[...].astype(jnp.bfloat16)
    ye = lax.dot_general(we_ref[...], x, _DN_NT,
                         preferred_element_type=jnp.float32)
    yo = lax.dot_general(wo_ref[...], x, _DN_NT,
                         preferred_element_type=jnp.float32)
    pk_ref[...] = _pack_rows(ye, yo)
    r_ref[...] = lax.dot_general(wroot_ref[...], x, _DN_NT,
                                 preferred_element_type=jnp.float32) + b_ref[...]


def _dense2_body(we_ref, wo_ref, wroot_ref, b_ref, agg_ref, r_ref,
                 pk_ref, r2_ref):
    h = jnp.maximum(agg_ref[...] + r_ref[...], 0.0).astype(jnp.bfloat16)
    ye = lax.dot_general(we_ref[...], h, _DN_NN,
                         preferred_element_type=jnp.float32)
    yo = lax.dot_general(wo_ref[...], h, _DN_NN,
                         preferred_element_type=jnp.float32)
    pk_ref[...] = _pack_rows(ye, yo)
    r2_ref[...] = lax.dot_general(wroot_ref[...], h, _DN_NN,
                                  preferred_element_type=jnp.float32) + b_ref[...]


def _pool_mlp_body(wp_ref, bp_ref, wf_ref, bf_ref, agg_ref, r_ref, batch_ref,
                   o_ref, sums_ref, cnt_ref):
    k = pl.program_id(0)

    @pl.when(k == 0)
    def _():
        sums_ref[...] = jnp.zeros_like(sums_ref)
        cnt_ref[...] = jnp.zeros_like(cnt_ref)

    h = jnp.maximum(agg_ref[...] + r_ref[...], 0.0)
    bm = batch_ref[0]
    h = jnp.where(bm >= 0, h, 0.0)
    b = bm.reshape(BN, 1)
    gids = lax.broadcasted_iota(jnp.int32, (BN, G), 1)
    oh = jnp.where(b == gids, 1.0, 0.0)
    sums_ref[...] += lax.dot_general(h, oh, _DN_NN,
                                     preferred_element_type=jnp.float32)
    cnt_ref[...] += jnp.sum(oh, axis=0, keepdims=True)

    @pl.when(k == NB - 1)
    def _():
        pooled = sums_ref[...] / jnp.maximum(cnt_ref[...], 1.0)
        z = lax.dot_general(wp_ref[...], pooled, _DN_NN,
                            preferred_element_type=jnp.float32) + bp_ref[...]
        z = jnp.maximum(z, 0.0)
        o = lax.dot_general(wf_ref[...], z, (((0,), (0,)), ((), ())),
                            preferred_element_type=jnp.float32) + bf_ref[...]
        o_ref[...] = jnp.broadcast_to(1.0 / (1.0 + jnp.exp(-o)), (8, G))


def _full(shape):
    return pl.BlockSpec(shape, lambda i: tuple(0 for _ in shape))


_dense1 = pl.pallas_call(
    _dense1_body,
    grid=(NB,),
    in_specs=[
        _full((HP, H)), _full((HP, H)), _full((H, H)), _full((H, 1)),
        pl.BlockSpec((BN, H), lambda i: (i, 0)),
    ],
    out_specs=[
        pl.BlockSpec((HP, BN), lambda i: (0, i)),
        pl.BlockSpec((H, BN), lambda i: (0, i)),
    ],
    out_shape=[
        jax.ShapeDtypeStruct((HP, NPAD), jnp.int32),
        jax.ShapeDtypeStruct((H, NPAD), jnp.float32),
    ],
)

_dense2 = pl.pallas_call(
    _dense2_body,
    grid=(NB,),
    in_specs=[
        _full((HP, H)), _full((HP, H)), _full((H, H)), _full((H, 1)),
        pl.BlockSpec((H, BN), lambda i: (0, i)),
        pl.BlockSpec((H, BN), lambda i: (0, i)),
    ],
    out_specs=[
        pl.BlockSpec((HP, BN), lambda i: (0, i)),
        pl.BlockSpec((H, BN), lambda i: (0, i)),
    ],
    out_shape=[
        jax.ShapeDtypeStruct((HP, NPAD), jnp.int32),
        jax.ShapeDtypeStruct((H, NPAD), jnp.float32),
    ],
)

_pool_mlp = pl.pallas_call(
    _pool_mlp_body,
    grid=(NB,),
    in_specs=[
        _full((H, H)), _full((H, 1)), _full((H, 1)), _full((1, 1)),
        pl.BlockSpec((H, BN), lambda i: (0, i)),
        pl.BlockSpec((H, BN), lambda i: (0, i)),
        pl.BlockSpec((1, 1, BN), lambda i: (i, 0, 0)),
    ],
    out_specs=pl.BlockSpec((8, G), lambda i: (0, 0)),
    out_shape=jax.ShapeDtypeStruct((8, G), jnp.float32),
    scratch_shapes=[
        pltpu.VMEM((H, G), jnp.float32),
        pltpu.VMEM((1, G), jnp.float32),
    ],
)


def _make_segsum():
    mesh = plsc.VectorSubcoreMesh(core_axis_name="c", subcore_axis_name="s")

    @functools.partial(
        pl.kernel,
        out_type=jax.ShapeDtypeStruct((H, NPAD), jnp.float32),
        mesh=mesh,
        compiler_params=pltpu.CompilerParams(
            use_tc_tiling_on_sc=False, needs_layout_passes=False),
        scratch_types=[
            pltpu.VMEM((RPK, NPAD), jnp.int32),
            pltpu.VMEM((2 * RPK, NPAD), jnp.float32),
            pltpu.VMEM((2, CHUNK), jnp.int32),
            pltpu.VMEM((2, CHUNK), jnp.int32),
            pltpu.SemaphoreType.DMA,
            pltpu.SemaphoreType.DMA,
            pltpu.SemaphoreType.DMA,
        ],
    )
    def seg(pk_hbm, e_hbm, out_hbm,
            table, acc, e_a, e_b, sem_a, sem_b, sem_tb):
        cid = lax.axis_index("c")
        sid = lax.axis_index("s")
        wid = sid * 2 + cid

        def start(g, ebuf, sem):
            pltpu.async_copy(e_hbm.at[g], ebuf, sem)

        def wait(ebuf, sem):
            pltpu.make_async_copy(e_hbm.at[0], ebuf, sem).wait()

        def process(ebuf):
            @plsc.parallel_loop(0, CHUNK, step=16, unroll=4)
            def gbody(base):
                sd = ebuf[0, pl.ds(base, 16)]
                sv = sd & 16383
                dv = (sd >> 14) & 16383
                wv = plsc.bitcast(ebuf[1, pl.ds(base, 16)], jnp.float32)
                for c in range(RPK):
                    cv = jnp.full((16,), c, jnp.int32)
                    gw = plsc.load_gather(table, [cv, sv])
                    bf = plsc.bitcast(gw, jnp.bfloat16)
                    lo, hi = plsc.unpack(bf, format=plsc.PackFormat.INTERLEAVED,
                                         preferred_element_type=jnp.float32)
                    plsc.addupdate_scatter(
                        acc, [jnp.full((16,), 2 * c, jnp.int32), dv], lo * wv)
                    plsc.addupdate_scatter(
                        acc, [jnp.full((16,), 2 * c + 1, jnp.int32), dv], hi * wv)

        r0 = wid * RPK
        tbl_cp = pltpu.async_copy(pk_hbm.at[pl.ds(r0, RPK), :], table, sem_tb)

        zeros16 = jnp.zeros((16,), jnp.float32)

        @plsc.parallel_loop(0, NPAD, step=32, unroll=4)
        def zbody(i):
            for r in range(2 * RPK):
                for u in range(2):
                    acc[r, pl.ds(i + u * 16, 16)] = zeros16

        tbl_cp.wait()

        start(0, e_a, sem_a)
        start(1, e_b, sem_b)

        def chunk_body(g2, carry):
            ga = g2 * 2
            wait(e_a, sem_a)
            process(e_a)

            @pl.when(ga + 2 < NCH)
            def _():
                start(ga + 2, e_a, sem_a)

            wait(e_b, sem_b)
            process(e_b)

            @pl.when(ga + 3 < NCH)
            def _():
                start(ga + 3, e_b, sem_b)
            return carry

        lax.fori_loop(0, NCH // 2, chunk_body, 0)

        pltpu.sync_copy(acc, out_hbm.at[pl.ds(wid * 2 * RPK, 2 * RPK), :])

    return seg


@functools.cache
def _get_segsum():
    return _make_segsum()


def kernel(x_static_graph, edge_index, edge_weight, batch, target_index,
           W1_rel, b1, W1_root, W2_rel, b2, W2_root, Wp, bp, Wf, bf):
    batch_pad = jnp.pad(batch, (0, NPAD - N), constant_values=-1)
    batch3d = batch_pad.reshape(NB, 1, BN)
    sd = edge_index[0] | (edge_index[1] << 14)
    wb = lax.bitcast_convert_type(edge_weight, jnp.int32)
    edges = jnp.stack([sd.reshape(NCH, CHUNK), wb.reshape(NCH, CHUNK)],
                      axis=1)

    segsum = _get_segsum()
    bft = jnp.bfloat16
    y1pk, r1 = _dense1(W1_rel[0::2].astype(bft), W1_rel[1::2].astype(bft),
                       W1_root.astype(bft), b1.reshape(H, 1), x_static_graph)
    agg1 = segsum(y1pk, edges)
    y2pk, r2 = _dense2(W2_rel[0::2].astype(bft), W2_rel[1::2].astype(bft),
                       W2_root.astype(bft), b2.reshape(H, 1), agg1, r1)
    agg2 = segsum(y2pk, edges)
    o = _pool_mlp(Wp, bp.reshape(H, 1), Wf.reshape(H, 1), bf.reshape(1, 1),
                  agg2, r2, batch3d)
    return o[0:1, :].reshape(G, 1)
